# baseline (speedup 1.0000x reference)
import functools

import jax
import jax.numpy as jnp
from jax import lax
from jax.experimental import pallas as pl
from jax.experimental.pallas import tpu as pltpu
from jax.experimental.pallas import tpu_sc as plsc

_GATHER_DNUMS = lax.GatherDimensionNumbers(
    offset_dims=(), collapsed_slice_dims=(0,), start_index_map=(0,))


def _shuffle(v, perm):
    return lax.gather(v, perm[:, None], _GATHER_DNUMS, slice_sizes=(1,),
                      mode=lax.GatherScatterMode.PROMISE_IN_BOUNDS)


L = 16
NC = 2
NS = 16
NW = NC * NS
CHUNK = 64


def _make_kernel(B, D, V):
    n_chunks = B // (NW * CHUNK)
    b_per_w = n_chunks * CHUNK
    d_vecs = D // L

    mesh = plsc.VectorSubcoreMesh(core_axis_name="c", subcore_axis_name="s")

    @functools.partial(
        pl.kernel,
        mesh=mesh,
        out_type=jax.ShapeDtypeStruct((B,), jnp.float32),
        scratch_types=[
            pltpu.VMEM((b_per_w,), jnp.int32),
            pltpu.VMEM((b_per_w,), jnp.int32),
            pltpu.VMEM((2, CHUNK, D), jnp.float32),
            pltpu.VMEM((2, CHUNK, D), jnp.float32),
            pltpu.VMEM((2, CHUNK, D), jnp.float32),
            pltpu.VMEM((b_per_w,), jnp.float32),
            pltpu.SemaphoreType.DMA((2,)),
            pltpu.SemaphoreType.DMA((2,)),
            pltpu.SemaphoreType.DMA((2,)),
        ],
    )
    def triplet_kernel(mention_hbm, h_hbm, t_hbm, table_hbm, out_hbm,
                       hidx, tidx, hrows, trows, mrows, outv,
                       hsem, tsem, msem):
        wid = lax.axis_index("s") * NC + lax.axis_index("c")
        iot = lax.iota(jnp.int32, L)
        base = wid * b_per_w

        HALF = CHUNK // 2

        def fire_mention(c, p):
            return pltpu.async_copy(
                mention_hbm.at[pl.ds(base + c * CHUNK, CHUNK)],
                mrows.at[p], msem.at[p])

        def fire_gathers(c, p):
            for half in range(2):
                pltpu.async_copy(
                    table_hbm.at[hidx.at[pl.ds(c * CHUNK + half * HALF,
                                               HALF)]],
                    hrows.at[p].at[pl.ds(half * HALF, HALF)], hsem.at[p])
                pltpu.async_copy(
                    table_hbm.at[tidx.at[pl.ds(c * CHUNK + half * HALF,
                                               HALF)]],
                    trows.at[p].at[pl.ds(half * HALF, HALF)], tsem.at[p])

        def fire(c, p):
            fire_mention(c, p)
            fire_gathers(c, p)

        def wait(c, p):
            pltpu.make_async_copy(
                table_hbm.at[hidx.at[pl.ds(c * CHUNK, CHUNK)]],
                hrows.at[p], hsem.at[p]).wait()
            pltpu.make_async_copy(
                table_hbm.at[tidx.at[pl.ds(c * CHUNK, CHUNK)]],
                trows.at[p], tsem.at[p]).wait()
            pltpu.make_async_copy(
                mention_hbm.at[pl.ds(base + c * CHUNK, CHUNK)],
                mrows.at[p], msem.at[p]).wait()

        fire_mention(0, 0)
        hicp = pltpu.async_copy(h_hbm.at[pl.ds(base, b_per_w)], hidx,
                                hsem.at[1])
        ticp = pltpu.async_copy(t_hbm.at[pl.ds(base, b_per_w)], tidx,
                                tsem.at[1])
        hicp.wait()
        ticp.wait()

        def compute(c, p):
            def group_body(g, carry):
                def sub_body(k, tot):
                    for rr in range(2):
                        r = k * 2 + rr
                        row = g * L + r
                        acc = jnp.zeros((L,), jnp.float32)
                        for j in range(d_vecs):
                            hv = hrows[p, row, pl.ds(j * L, L)]
                            tv = trows[p, row, pl.ds(j * L, L)]
                            mv = mrows[p, row, pl.ds(j * L, L)]
                            d = (hv + mv) - tv
                            acc = acc + d * d
                        for sh in (8, 4, 2, 1):
                            acc = acc + _shuffle(acc, iot ^ sh)
                        tot = jnp.where(iot == r, acc, tot)
                    return tot

                tot = lax.fori_loop(0, 8, sub_body,
                                    jnp.zeros((L,), jnp.float32))
                x = jnp.clip(tot, jnp.float32(2.0 ** -24),
                             jnp.float32(2.0 ** 40))
                u = x * jnp.float32(2.0 ** 24)
                yac = jnp.full((L,), 2.0 ** 12, jnp.float32)
                for s in (32, 16, 8, 4, 2, 1):
                    cond = u >= jnp.float32(2.0 ** s)
                    u = jnp.where(cond, u * jnp.float32(2.0 ** -s), u)
                    yac = jnp.where(
                        cond, yac * jnp.float32(2.0 ** (-s / 2)), yac)
                y = yac * (jnp.float32(1.2989) - jnp.float32(0.2929) * u)
                for _ in range(3):
                    y = y * (jnp.float32(1.5)
                             - jnp.float32(0.5) * x * y * y)
                outv[pl.ds(c * CHUNK + g * L, L)] = -(x * y)
                return carry

            lax.fori_loop(0, CHUNK // L, group_body, 0)

        fire_gathers(0, 0)

        def chunk_body(c, carry):
            p = c & 1

            @pl.when(c + 1 < n_chunks)
            def _():
                fire(c + 1, 1 - p)

            wait(c, p)
            compute(c, p)
            return carry

        lax.fori_loop(0, n_chunks, chunk_body, 0)

        pltpu.sync_copy(outv, out_hbm.at[pl.ds(base, b_per_w)])

    return triplet_kernel


def kernel(mention, h, t, emb_table):
    B, D = mention.shape
    V = emb_table.shape[0]
    return _make_kernel(B, D, V)(mention, h, t, emb_table)


# speedup vs baseline: 1.0501x; 1.0501x over previous
import functools

import jax
import jax.numpy as jnp
from jax import lax
from jax.experimental import pallas as pl
from jax.experimental.pallas import tpu as pltpu
from jax.experimental.pallas import tpu_sc as plsc

_GATHER_DNUMS = lax.GatherDimensionNumbers(
    offset_dims=(), collapsed_slice_dims=(0,), start_index_map=(0,))


def _shuffle(v, perm):
    return lax.gather(v, perm[:, None], _GATHER_DNUMS, slice_sizes=(1,),
                      mode=lax.GatherScatterMode.PROMISE_IN_BOUNDS)


L = 16
NC = 2
NS = 16
NW = NC * NS
CHUNK = 64


def _make_kernel(B, D, V):
    n_chunks = B // (NW * CHUNK)
    b_per_w = n_chunks * CHUNK
    d_vecs = D // L

    mesh = plsc.VectorSubcoreMesh(core_axis_name="c", subcore_axis_name="s")

    @functools.partial(
        pl.kernel,
        mesh=mesh,
        out_type=jax.ShapeDtypeStruct((B,), jnp.float32),
        scratch_types=[
            pltpu.VMEM((b_per_w,), jnp.int32),
            pltpu.VMEM((b_per_w,), jnp.int32),
            pltpu.VMEM((4, CHUNK, D), jnp.float32),
            pltpu.VMEM((4, CHUNK, D), jnp.float32),
            pltpu.VMEM((4, CHUNK, D), jnp.float32),
            pltpu.VMEM((b_per_w,), jnp.float32),
            pltpu.SemaphoreType.DMA((4,)),
            pltpu.SemaphoreType.DMA((4,)),
            pltpu.SemaphoreType.DMA((4,)),
        ],
    )
    def triplet_kernel(mention_hbm, h_hbm, t_hbm, table_hbm, out_hbm,
                       hidx, tidx, hrows, trows, mrows, outv,
                       hsem, tsem, msem):
        wid = lax.axis_index("s") * NC + lax.axis_index("c")
        iot = lax.iota(jnp.int32, L)
        base = wid * b_per_w

        HALF = CHUNK // 2

        def fire_mention(c, p):
            return pltpu.async_copy(
                mention_hbm.at[pl.ds(base + c * CHUNK, CHUNK)],
                mrows.at[p], msem.at[p])

        def fire_gathers(c, p):
            for half in range(2):
                pltpu.async_copy(
                    table_hbm.at[hidx.at[pl.ds(c * CHUNK + half * HALF,
                                               HALF)]],
                    hrows.at[p].at[pl.ds(half * HALF, HALF)], hsem.at[p])
                pltpu.async_copy(
                    table_hbm.at[tidx.at[pl.ds(c * CHUNK + half * HALF,
                                               HALF)]],
                    trows.at[p].at[pl.ds(half * HALF, HALF)], tsem.at[p])

        def fire(c, p):
            fire_mention(c, p)
            fire_gathers(c, p)

        def wait(c, p):
            pltpu.make_async_copy(
                table_hbm.at[hidx.at[pl.ds(c * CHUNK, CHUNK)]],
                hrows.at[p], hsem.at[p]).wait()
            pltpu.make_async_copy(
                table_hbm.at[tidx.at[pl.ds(c * CHUNK, CHUNK)]],
                trows.at[p], tsem.at[p]).wait()
            pltpu.make_async_copy(
                mention_hbm.at[pl.ds(base + c * CHUNK, CHUNK)],
                mrows.at[p], msem.at[p]).wait()

        fire_mention(0, 0)
        hicp = pltpu.async_copy(h_hbm.at[pl.ds(base, b_per_w)], hidx,
                                hsem.at[1])
        ticp = pltpu.async_copy(t_hbm.at[pl.ds(base, b_per_w)], tidx,
                                tsem.at[1])
        hicp.wait()
        ticp.wait()

        def compute(c, p):
            def group_body(g, carry):
                def sub_body(k, tot):
                    for rr in range(2):
                        r = k * 2 + rr
                        row = g * L + r
                        acc = jnp.zeros((L,), jnp.float32)
                        for j in range(d_vecs):
                            hv = hrows[p, row, pl.ds(j * L, L)]
                            tv = trows[p, row, pl.ds(j * L, L)]
                            mv = mrows[p, row, pl.ds(j * L, L)]
                            d = (hv + mv) - tv
                            acc = acc + d * d
                        for sh in (8, 4, 2, 1):
                            acc = acc + _shuffle(acc, iot ^ sh)
                        tot = jnp.where(iot == r, acc, tot)
                    return tot

                tot = lax.fori_loop(0, 8, sub_body,
                                    jnp.zeros((L,), jnp.float32))
                x = jnp.clip(tot, jnp.float32(2.0 ** -24),
                             jnp.float32(2.0 ** 40))
                u = x * jnp.float32(2.0 ** 24)
                yac = jnp.full((L,), 2.0 ** 12, jnp.float32)
                for s in (32, 16, 8, 4, 2, 1):
                    cond = u >= jnp.float32(2.0 ** s)
                    u = jnp.where(cond, u * jnp.float32(2.0 ** -s), u)
                    yac = jnp.where(
                        cond, yac * jnp.float32(2.0 ** (-s / 2)), yac)
                y = yac * (jnp.float32(1.2989) - jnp.float32(0.2929) * u)
                for _ in range(3):
                    y = y * (jnp.float32(1.5)
                             - jnp.float32(0.5) * x * y * y)
                outv[pl.ds(c * CHUNK + g * L, L)] = -(x * y)
                return carry

            lax.fori_loop(0, CHUNK // L, group_body, 0)

        fire_gathers(0, 0)
        fire(1, 1)

        def chunk_body(c, carry):
            @pl.when(c + 2 < n_chunks)
            def _():
                fire(c + 2, (c + 2) & 3)

            wait(c, c & 3)
            compute(c, c & 3)
            return carry

        lax.fori_loop(0, n_chunks, chunk_body, 0)

        pltpu.sync_copy(outv, out_hbm.at[pl.ds(base, b_per_w)])

    return triplet_kernel


def kernel(mention, h, t, emb_table):
    B, D = mention.shape
    V = emb_table.shape[0]
    return _make_kernel(B, D, V)(mention, h, t, emb_table)


# speedup vs baseline: 1.0593x; 1.0087x over previous
import functools

import jax
import jax.numpy as jnp
from jax import lax
from jax.experimental import pallas as pl
from jax.experimental.pallas import tpu as pltpu
from jax.experimental.pallas import tpu_sc as plsc

_GATHER_DNUMS = lax.GatherDimensionNumbers(
    offset_dims=(), collapsed_slice_dims=(0,), start_index_map=(0,))


def _shuffle(v, perm):
    return lax.gather(v, perm[:, None], _GATHER_DNUMS, slice_sizes=(1,),
                      mode=lax.GatherScatterMode.PROMISE_IN_BOUNDS)


L = 16
NC = 2
NS = 16
NW = NC * NS
CHUNK = 64


def _make_kernel(B, D, V):
    n_chunks = B // (NW * CHUNK)
    b_per_w = n_chunks * CHUNK
    d_vecs = D // L

    mesh = plsc.VectorSubcoreMesh(core_axis_name="c", subcore_axis_name="s")

    @functools.partial(
        pl.kernel,
        mesh=mesh,
        out_type=jax.ShapeDtypeStruct((B,), jnp.float32),
        scratch_types=[
            pltpu.VMEM((b_per_w,), jnp.int32),
            pltpu.VMEM((b_per_w,), jnp.int32),
            pltpu.VMEM((4, CHUNK, D), jnp.float32),
            pltpu.VMEM((4, CHUNK, D), jnp.float32),
            pltpu.VMEM((4, CHUNK, D), jnp.float32),
            pltpu.VMEM((b_per_w,), jnp.float32),
            pltpu.SemaphoreType.DMA((4,)),
            pltpu.SemaphoreType.DMA((4,)),
            pltpu.SemaphoreType.DMA((4,)),
        ],
    )
    def triplet_kernel(mention_hbm, h_hbm, t_hbm, table_hbm, out_hbm,
                       hidx, tidx, hrows, trows, mrows, outv,
                       hsem, tsem, msem):
        wid = lax.axis_index("s") * NC + lax.axis_index("c")
        iot = lax.iota(jnp.int32, L)
        base = wid * b_per_w

        HALF = CHUNK // 2

        def fire_mention(c, p):
            return pltpu.async_copy(
                mention_hbm.at[pl.ds(base + c * CHUNK, CHUNK)],
                mrows.at[p], msem.at[p])

        def fire_gathers(c, p):
            for half in range(2):
                pltpu.async_copy(
                    table_hbm.at[hidx.at[pl.ds(c * CHUNK + half * HALF,
                                               HALF)]],
                    hrows.at[p].at[pl.ds(half * HALF, HALF)], hsem.at[p])
                pltpu.async_copy(
                    table_hbm.at[tidx.at[pl.ds(c * CHUNK + half * HALF,
                                               HALF)]],
                    trows.at[p].at[pl.ds(half * HALF, HALF)], tsem.at[p])

        def fire(c, p):
            fire_mention(c, p)
            fire_gathers(c, p)

        def wait(c, p):
            pltpu.make_async_copy(
                table_hbm.at[hidx.at[pl.ds(c * CHUNK, CHUNK)]],
                hrows.at[p], hsem.at[p]).wait()
            pltpu.make_async_copy(
                table_hbm.at[tidx.at[pl.ds(c * CHUNK, CHUNK)]],
                trows.at[p], tsem.at[p]).wait()
            pltpu.make_async_copy(
                mention_hbm.at[pl.ds(base + c * CHUNK, CHUNK)],
                mrows.at[p], msem.at[p]).wait()

        fire_mention(0, 0)
        hicp = pltpu.async_copy(h_hbm.at[pl.ds(base, b_per_w)], hidx,
                                hsem.at[1])
        ticp = pltpu.async_copy(t_hbm.at[pl.ds(base, b_per_w)], tidx,
                                tsem.at[1])
        hicp.wait()
        ticp.wait()

        def compute(c, p):
            def group_body(g, carry):
                def sub_body(k, tot):
                    for rr in range(2):
                        r = k * 2 + rr
                        row = g * L + r
                        acc = jnp.zeros((L,), jnp.float32)
                        for j in range(d_vecs):
                            hv = hrows[p, row, pl.ds(j * L, L)]
                            tv = trows[p, row, pl.ds(j * L, L)]
                            mv = mrows[p, row, pl.ds(j * L, L)]
                            d = (hv + mv) - tv
                            acc = acc + d * d
                        for sh in (8, 4, 2, 1):
                            acc = acc + _shuffle(acc, iot ^ sh)
                        tot = jnp.where(iot == r, acc, tot)
                    return tot

                tot = lax.fori_loop(0, 8, sub_body,
                                    jnp.zeros((L,), jnp.float32))
                x = jnp.clip(tot, jnp.float32(2.0 ** -24),
                             jnp.float32(2.0 ** 40))
                u = x * jnp.float32(2.0 ** 24)
                yac = jnp.full((L,), 2.0 ** 12, jnp.float32)
                for s in (32, 16, 8, 4, 2, 1):
                    cond = u >= jnp.float32(2.0 ** s)
                    u = jnp.where(cond, u * jnp.float32(2.0 ** -s), u)
                    yac = jnp.where(
                        cond, yac * jnp.float32(2.0 ** (-s / 2)), yac)
                y = yac * (jnp.float32(1.2989) - jnp.float32(0.2929) * u)
                for _ in range(3):
                    y = y * (jnp.float32(1.5)
                             - jnp.float32(0.5) * x * y * y)
                outv[pl.ds(c * CHUNK + g * L, L)] = -(x * y)
                return carry

            lax.fori_loop(0, CHUNK // L, group_body, 0)

        fire_gathers(0, 0)
        fire(1, 1)
        fire(2, 2)

        def chunk_body(c, carry):
            @pl.when(c + 3 < n_chunks)
            def _():
                fire(c + 3, (c + 3) & 3)

            wait(c, c & 3)
            compute(c, c & 3)
            return carry

        lax.fori_loop(0, n_chunks, chunk_body, 0)

        pltpu.sync_copy(outv, out_hbm.at[pl.ds(base, b_per_w)])

    return triplet_kernel


def kernel(mention, h, t, emb_table):
    B, D = mention.shape
    V = emb_table.shape[0]
    return _make_kernel(B, D, V)(mention, h, t, emb_table)
